# baseline (speedup 1.0000x reference)
import functools

import jax
import jax.numpy as jnp
from jax import lax
from jax.experimental import pallas as pl
from jax.experimental.pallas import tpu as pltpu
from jax.experimental.pallas import tpu_sc as plsc

U = 100000
I = 100000
E = 64
B = 16384
NC_CLIENTS = 3

CHUNK = 128
NUM_CHUNKS = B // CHUNK


def _gather_body(nchunks_per_worker, num_cores,
                 users_hbm, pos_hbm, neg_hbm, wu, wi, c0, c1, c2,
                 out_u, out_c0, out_c1, out_c2, out_p, out_n,
                 idx_v, rows_v, sem):
  wid = lax.axis_index("s") * num_cores + lax.axis_index("c")
  row0 = wid * nchunks_per_worker
  for j in range(nchunks_per_worker):
    crow = row0 + j
    base = crow * CHUNK
    pltpu.sync_copy(users_hbm.at[crow], idx_v)
    for table, out in ((wu, out_u), (c0, out_c0), (c1, out_c1), (c2, out_c2)):
      pltpu.async_copy(table.at[idx_v], rows_v, sem).wait()
      pltpu.sync_copy(rows_v, out.at[pl.ds(base, CHUNK)])
    pltpu.sync_copy(pos_hbm.at[crow], idx_v)
    pltpu.async_copy(wi.at[idx_v], rows_v, sem).wait()
    pltpu.sync_copy(rows_v, out_p.at[pl.ds(base, CHUNK)])
    pltpu.sync_copy(neg_hbm.at[crow], idx_v)
    pltpu.async_copy(wi.at[idx_v], rows_v, sem).wait()
    pltpu.sync_copy(rows_v, out_n.at[pl.ds(base, CHUNK)])


def _sc_gather(users, pos_items, neg_items, wu, wi, c0, c1, c2):
  info = plsc.get_sparse_core_info()
  num_cores, num_subcores = info.num_cores, info.num_subcores
  nw = num_cores * num_subcores
  nchunks_per_worker = NUM_CHUNKS // nw

  mesh = plsc.VectorSubcoreMesh(core_axis_name="c", subcore_axis_name="s")
  out_t = [jax.ShapeDtypeStruct((B, E), jnp.float32)] * 6
  scratch = [
      pltpu.VMEM((CHUNK,), jnp.int32),
      pltpu.VMEM((CHUNK, E), jnp.float32),
      pltpu.SemaphoreType.DMA,
  ]
  users2 = users.astype(jnp.int32).reshape(NUM_CHUNKS, CHUNK)
  pos2 = pos_items.astype(jnp.int32).reshape(NUM_CHUNKS, CHUNK)
  neg2 = neg_items.astype(jnp.int32).reshape(NUM_CHUNKS, CHUNK)
  body = functools.partial(_gather_body, nchunks_per_worker, num_cores)
  return pl.kernel(
      body, out_type=out_t, mesh=mesh, scratch_types=scratch,
      compiler_params=pltpu.CompilerParams(use_tc_tiling_on_sc=False))(
      users2, pos2, neg2, wu, wi, c0, c1, c2)


def _tc_body(uu, c0r, c1r, c2r, pr, nr,
             w_dnn, b_dnn, w_di, b_di, wd0, bd0, wd1, bd1, wd2, bd2,
             out_ref):
  f32 = jnp.float32
  ucat = jnp.concatenate(
      [uu[...], c0r[...], c1r[...], c2r[...]], axis=1)
  server = jax.nn.relu(
      jnp.dot(ucat, w_dnn[...], preferred_element_type=f32) + b_dnn[...])
  ep = jax.nn.relu(
      jnp.dot(pr[...], w_di[...], preferred_element_type=f32) + b_di[...])
  en = jax.nn.relu(
      jnp.dot(nr[...], w_di[...], preferred_element_type=f32) + b_di[...])
  d0 = jax.nn.relu(
      jnp.dot(c0r[...], wd0[...], preferred_element_type=f32) + bd0[...])
  d1 = jax.nn.relu(
      jnp.dot(c1r[...], wd1[...], preferred_element_type=f32) + bd1[...])
  d2 = jax.nn.relu(
      jnp.dot(c2r[...], wd2[...], preferred_element_type=f32) + bd2[...])
  ps = jnp.zeros_like(ep[:, 0])
  ns = jnp.zeros_like(ps)
  for k, eu in enumerate((server, d0, d1, d2)):
    ps = ps + jnp.sum(eu * ep, axis=1)
    ns = ns + jnp.sum(eu * en, axis=1)
    out_ref[k, :] = ps
    out_ref[4 + k, :] = ns


def _tc_compute(uu, c0r, c1r, c2r, pr, nr,
                w_dnn, b_dnn, w_di, b_di, wd0, bd0, wd1, bd1, wd2, bd2):
  blk = 2048
  grid = (B // blk,)
  row_spec = pl.BlockSpec((blk, E), lambda i: (i, 0))
  full = lambda shape: pl.BlockSpec(shape, lambda i: (0,) * len(shape))
  in_specs = [row_spec] * 6 + [
      full((4 * E, E)), full((1, E)),
      full((E, E)), full((1, E)),
      full((E, E)), full((1, E)),
      full((E, E)), full((1, E)),
      full((E, E)), full((1, E)),
  ]
  out_spec = pl.BlockSpec((8, blk), lambda i: (0, i))
  return pl.pallas_call(
      _tc_body,
      grid=grid,
      in_specs=in_specs,
      out_specs=out_spec,
      out_shape=jax.ShapeDtypeStruct((8, B), jnp.float32),
  )(uu, c0r, c1r, c2r, pr, nr,
    w_dnn, b_dnn.reshape(1, E), w_di, b_di.reshape(1, E),
    wd0, bd0.reshape(1, E), wd1, bd1.reshape(1, E), wd2, bd2.reshape(1, E))


def kernel(users, pos_items, neg_items, W_user, W_item, C0, C1, C2,
           W_dnn, b_dnn, W_di, b_di, Wd0, bd0, Wd1, bd1, Wd2, bd2):
  uu, c0r, c1r, c2r, pr, nr = _sc_gather(
      users, pos_items, neg_items, W_user, W_item, C0, C1, C2)
  scores = _tc_compute(uu, c0r, c1r, c2r, pr, nr,
                       W_dnn, b_dnn, W_di, b_di, Wd0, bd0, Wd1, bd1, Wd2, bd2)
  pos_list = scores[0:4]
  neg_list = scores[4:8]
  return (pos_list[3], neg_list[3], pos_list, neg_list)
